# Initial kernel scaffold; baseline (speedup 1.0000x reference)
import jax
import jax.numpy as jnp
from jax.experimental import pallas as pl


def kernel(x, weight):
    raise NotImplementedError("write your pallas kernel here")


# speedup vs baseline: 1.1092x; 1.1092x over previous
import functools

import jax
import jax.numpy as jnp
from jax import lax
from jax.experimental import pallas as pl
from jax.experimental.pallas import tpu as pltpu
from jax.experimental.pallas import tpu_sc as plsc

NUM_ROWS = 16384 * 50
DIM = 32

_NC = 2
_NS = 16
_NW = _NC * _NS
_BPW = NUM_ROWS // _NW
_CHUNK = 1600
_NCHUNK = _BPW // _CHUNK


def _make_gather():
    mesh = plsc.VectorSubcoreMesh(core_axis_name="c", subcore_axis_name="s")

    @functools.partial(
        pl.kernel,
        mesh=mesh,
        out_type=jax.ShapeDtypeStruct((NUM_ROWS, DIM), jnp.float32),
        compiler_params=pltpu.CompilerParams(use_tc_tiling_on_sc=False),
        scratch_types=[
            pltpu.VMEM((_CHUNK,), jnp.int32),
            pltpu.VMEM((_CHUNK,), jnp.int32),
            pltpu.VMEM((_CHUNK, DIM), jnp.float32),
            pltpu.VMEM((_CHUNK, DIM), jnp.float32),
            pltpu.SemaphoreType.DMA,
            pltpu.SemaphoreType.DMA,
            pltpu.SemaphoreType.DMA,
            pltpu.SemaphoreType.DMA,
        ],
    )
    def gather_kernel(idx_hbm, tbl_hbm, out_hbm, idx0, idx1, rows0, rows1,
                      g0, g1, s0, s1):
        wid = lax.axis_index("s") * _NC + lax.axis_index("c")
        base = wid * _BPW
        idx_v = (idx0, idx1)
        rows_v = (rows0, rows1)
        gsem = (g0, g1)
        osem = (s0, s1)
        gathers = [None, None]
        stores = [None, None]
        for i in range(_NCHUNK):
            b = i % 2
            if stores[b] is not None:
                stores[b].wait()
            pltpu.sync_copy(idx_hbm.at[pl.ds(base + i * _CHUNK, _CHUNK)],
                            idx_v[b])
            gathers[b] = pltpu.async_copy(tbl_hbm.at[idx_v[b]],
                                          rows_v[b], gsem[b])
            if i >= 1:
                ob = 1 - b
                gathers[ob].wait()
                stores[ob] = pltpu.async_copy(
                    rows_v[ob],
                    out_hbm.at[pl.ds(base + (i - 1) * _CHUNK, _CHUNK)],
                    osem[ob])
        lb = (_NCHUNK - 1) % 2
        gathers[lb].wait()
        stores[lb] = pltpu.async_copy(
            rows_v[lb],
            out_hbm.at[pl.ds(base + (_NCHUNK - 1) * _CHUNK, _CHUNK)],
            osem[lb])
        stores[0].wait()
        stores[1].wait()

    return gather_kernel


_gather = _make_gather()


@jax.jit
def kernel(x, weight):
    flat_idx = x.reshape(-1).astype(jnp.int32)
    out = _gather(flat_idx, weight)
    return out.reshape(x.shape + (DIM,))


# speedup vs baseline: 2.4918x; 2.2464x over previous
import functools

import jax
import jax.numpy as jnp
from jax import lax
from jax.experimental import pallas as pl
from jax.experimental.pallas import tpu as pltpu
from jax.experimental.pallas import tpu_sc as plsc

B = 16384
S = 50
DIM = 32

_NC = 2
_NS = 16
_NW = _NC * _NS
_CB = B // _NW
_OUT_ELEMS = S * DIM * B


def _make_gather():
    mesh = plsc.VectorSubcoreMesh(core_axis_name="c", subcore_axis_name="s")

    @functools.partial(
        pl.kernel,
        mesh=mesh,
        out_type=jax.ShapeDtypeStruct((_OUT_ELEMS,), jnp.float32),
        compiler_params=pltpu.CompilerParams(use_tc_tiling_on_sc=False,
                                             needs_layout_passes=False),
        scratch_types=[
            pltpu.VMEM((_CB,), jnp.int32),
            pltpu.VMEM((_CB,), jnp.int32),
            pltpu.VMEM((_CB, DIM), jnp.float32),
            pltpu.VMEM((_CB, DIM), jnp.float32),
            pltpu.VMEM((_CB * DIM,), jnp.float32),
            pltpu.VMEM((_CB * DIM,), jnp.float32),
            pltpu.SemaphoreType.DMA,
            pltpu.SemaphoreType.DMA,
            pltpu.SemaphoreType.DMA,
            pltpu.SemaphoreType.DMA,
        ],
    )
    def gather_kernel(idx_hbm, tbl_hbm, out_hbm,
                      idx0, idx1, rows0, rows1, t0, t1, g0, g1, o0, o1):
        wid = lax.axis_index("s") * _NC + lax.axis_index("c")
        b0 = wid * _CB
        idx_v = (idx0, idx1)
        rows_v = (rows0, rows1)
        t_v = (t0, t1)
        gsem = (g0, g1)
        osem = (o0, o1)

        iota = lax.iota(jnp.int32, 16)
        pats = []
        for j in range(16):
            dv = (j + iota) & 15
            dstp = ((dv >> 3) << 12) + ((dv & 7) << 7) + iota
            pats.append((dv, dstp))

        def transpose_block(rows, t):
            def q_body(q):
                def i_body(i):
                    l0 = q * 128 + i * 16
                    db0 = (q << 10) + i * 16
                    lvec = l0 + iota
                    for d0 in (0, 16):
                        for j in range(16):
                            dv, dstp = pats[j]
                            v = plsc.load_gather(rows, [lvec, dv + d0])
                            plsc.store_scatter(
                                t, [dstp + (db0 + (d0 >> 4) * 8192)], v)
                pl.loop(0, 8)(i_body)
            pl.loop(0, 4)(q_body)

        def launch(s, bi):
            pltpu.sync_copy(idx_hbm.at[pl.ds(s * B + b0, _CB)], idx_v[bi])
            return pltpu.async_copy(tbl_hbm.at[idx_v[bi]], rows_v[bi],
                                    gsem[bi])

        def wait_gather(bi):
            pltpu.make_async_copy(tbl_hbm.at[idx_v[bi]], rows_v[bi],
                                  gsem[bi]).wait()

        def launch_stores(s, bi):
            base = s * 512 + wid * 4
            return [
                pltpu.async_copy(
                    t_v[bi].at[pl.ds(d8 * 4096, 4096)],
                    out_hbm.at[pl.ds((base + d8 * 128) * 1024, 4096)],
                    osem[bi])
                for d8 in range(4)
            ]

        def wait_stores(s, bi):
            base = s * 512 + wid * 4
            for d8 in range(4):
                pltpu.make_async_copy(
                    t_v[bi].at[pl.ds(d8 * 4096, 4096)],
                    out_hbm.at[pl.ds((base + d8 * 128) * 1024, 4096)],
                    osem[bi]).wait()

        launch(0, 0)

        def body(s2):
            for off in range(2):
                s = s2 + off
                bi = off
                if off == 0:
                    launch(s + 1, 1 - bi)
                else:
                    @pl.when(s2 < S - 2)
                    def _():
                        launch(s + 1, 1 - bi)
                wait_gather(bi)
                if off == 0:
                    @pl.when(s2 >= 2)
                    def _():
                        wait_stores(s - 2, bi)
                else:
                    @pl.when(s2 >= 1)
                    def _():
                        wait_stores(s - 2, bi)
                transpose_block(rows_v[bi], t_v[bi])
                launch_stores(s, bi)
        pl.loop(0, S, step=2)(body)

        wait_stores(S - 2, 0)
        wait_stores(S - 1, 1)

    return gather_kernel


_gather = _make_gather()


@jax.jit
def kernel(x, weight):
    xt = jnp.swapaxes(x, 0, 1).reshape(-1).astype(jnp.int32)
    flat = _gather(xt, weight)
    out5 = flat.reshape(S, 4, B // 128, 8, 128)
    return out5.transpose(2, 4, 0, 1, 3).reshape(B, S, DIM)
